# baseline (speedup 1.0000x reference)
import functools

import jax
import jax.numpy as jnp
from jax.experimental import pallas as pl
from jax.experimental.pallas import tpu as pltpu

BLK = 128
NB = 64
SEQ_CHUNK = 2048
KPAIRS = 1024
N_CHUNKS = 8192 // SEQ_CHUNK
NBC = SEQ_CHUNK // BLK


def _blockdiag(w, rows, cols, rep):
    row = jnp.concatenate([w] * rep, axis=1)
    tall = jnp.concatenate([row] * rep, axis=0)
    r = jax.lax.broadcasted_iota(jnp.int32, (rows, cols), 0)
    c = jax.lax.broadcasted_iota(jnp.int32, (rows, cols), 1)
    mask = (r // (rows // rep)) == (c // (cols // rep))
    return jnp.where(mask, tall, 0.0)


def _fused_kernel(q_ref, k_ref, W1_ref, b1_ref, W2_ref, b2_ref, W3_ref,
                  b3_ref, Wi1_ref, bi1_ref, Wi2_ref, bi2_ref,
                  out_ref, qa_s, ka_s):
    s = pl.program_id(1)

    def block_mean(x):
        t = jnp.sum(x.reshape(12, 64, NBC, BLK), axis=3) * (1.0 / BLK)
        return jnp.transpose(t, (2, 0, 1)).reshape(NBC, 768)

    qa_s[pl.ds(s * NBC, NBC), :] = block_mean(q_ref[0])
    ka_s[pl.ds(s * NBC, NBC), :] = block_mean(k_ref[0])

    @pl.when(s == N_CHUNKS - 1)
    def _score():
        W1d = _blockdiag(W1_ref[...].T, 768, 192, 12)
        W2d = _blockdiag(W2_ref[...].T, 192, 96, 12)
        W3d = _blockdiag(W3_ref[...].T, 96, 12, 12)
        wi1t = Wi1_ref[...].T
        WAd = _blockdiag(wi1t[:64], 768, 192, 12)
        WBd = _blockdiag(wi1t[64:], 768, 192, 12)
        Wd = _blockdiag(Wi2_ref[...].T, 192, 12, 12)
        b1d = jnp.concatenate([b1_ref[...][None, :]] * 12, axis=1)
        b2d = jnp.concatenate([b2_ref[...][None, :]] * 12, axis=1)
        b3d = jnp.concatenate([b3_ref[...][None, :]] * 12, axis=1)
        bi1d = jnp.concatenate([bi1_ref[...][None, :]] * 12, axis=1)
        bi2d = jnp.concatenate([bi2_ref[...][None, :]] * 12, axis=1)

        qa = qa_s[...]
        ka = ka_s[...]

        def imp(x):
            h1 = jax.nn.relu(jnp.dot(x, W1d) + b1d)
            h2 = jax.nn.relu(jnp.dot(h1, W2d) + b2d)
            return jax.nn.sigmoid(jnp.dot(h2, W3d) + b3d)

        q_imp = imp(qa)
        k_imp = imp(ka)

        a2 = jnp.dot(qa, WAd)
        b2m = jnp.dot(ka, WBd) + bi1d
        pre = jax.nn.relu(a2[:, None, :] + b2m[None, :, :])
        pre = pre.reshape(NB * NB, 192)
        inter = jax.nn.sigmoid(jnp.dot(pre, Wd) + bi2d)

        qrep = jnp.broadcast_to(q_imp[:, None, :], (NB, NB, 12)).reshape(NB * NB, 12)
        krep = jnp.broadcast_to(k_imp[None, :, :], (NB, NB, 12)).reshape(NB * NB, 12)
        out_ref[0] = inter * qrep * krep


def _topk_kernel(v_ref, out_ref):
    bits = jax.lax.bitcast_convert_type(v_ref[...], jnp.int32)
    t = jnp.zeros((48, 1), jnp.int32)
    for bit in range(30, -1, -1):
        cand = t | (1 << bit)
        cnt = jnp.sum((bits >= cand).astype(jnp.int32), axis=1, keepdims=True)
        t = jnp.where(cnt >= KPAIRS, cand, t)
    out_ref[...] = (bits >= t).astype(jnp.int8)


@functools.partial(jax.jit, static_argnames=())
def kernel(q, k, W1, b1, W2, b2, W3, b3, Wi1, bi1, Wi2, bi2):
    batch, seq, heads, hd = q.shape
    nb = seq // BLK

    qT = jnp.transpose(q, (0, 2, 3, 1))
    kT = jnp.transpose(k, (0, 2, 3, 1))

    full = lambda shape: pl.BlockSpec(shape, lambda b, s: tuple([0] * len(shape)))
    combined = pl.pallas_call(
        _fused_kernel,
        grid=(batch, N_CHUNKS),
        in_specs=[
            pl.BlockSpec((1, heads, hd, SEQ_CHUNK), lambda b, s: (b, 0, 0, s)),
            pl.BlockSpec((1, heads, hd, SEQ_CHUNK), lambda b, s: (b, 0, 0, s)),
            full(W1.shape), full(b1.shape), full(W2.shape), full(b2.shape),
            full(W3.shape), full(b3.shape), full(Wi1.shape), full(bi1.shape),
            full(Wi2.shape), full(bi2.shape),
        ],
        out_specs=pl.BlockSpec((1, nb * nb, heads), lambda b, s: (b, 0, 0)),
        out_shape=jax.ShapeDtypeStruct((batch, nb * nb, heads), jnp.float32),
        scratch_shapes=[
            pltpu.VMEM((nb, heads * hd), jnp.float32),
            pltpu.VMEM((nb, heads * hd), jnp.float32),
        ],
    )(qT, kT, W1, b1, W2, b2, W3, b3, Wi1, bi1, Wi2, bi2)

    groups = combined.reshape(batch * heads, nb * nb)
    mask8 = pl.pallas_call(
        _topk_kernel,
        grid=(1,),
        in_specs=[pl.BlockSpec(groups.shape, lambda i: (0, 0))],
        out_specs=pl.BlockSpec(groups.shape, lambda i: (0, 0)),
        out_shape=jax.ShapeDtypeStruct(groups.shape, jnp.int8),
    )(groups)

    return mask8.reshape(batch, heads, nb, nb).astype(bool)
